# baseline (speedup 1.0000x reference)
import functools

import jax
import jax.numpy as jnp
from jax import lax
from jax.experimental import pallas as pl
from jax.experimental.pallas import tpu as pltpu
from jax.experimental.pallas import tpu_sc as plsc

NUM_SPARSE = 26
NUM_DENSE = 13
VOCAB_ROWS = 100000
LATENT = 16
BATCH = 16384

NUM_CORES = 2
NUM_SUBCORES = 16
NW = NUM_CORES * NUM_SUBCORES
BPW = BATCH // NW
CHUNK = 128
NCHUNK = BPW // CHUNK

FCOLS = 4096
FG = 25
VP = FG * FCOLS

HALF0 = list(range(24))
HALF1 = list(range(24, 26))


def _make_fmt(n_tables, widths):
    n_groups = len(widths)

    def body(*refs):
        ins = refs[:n_tables]
        outs = refs[n_tables:]
        done = 0
        for g, w in enumerate(widths):
            k = w // 16
            x = jnp.concatenate([ins[done + i][...] for i in range(k)], axis=0)
            if w == 128:
                outs[g][...] = x.T
            else:
                outs[g][:, 0:w] = x.T
            done += k

    def call(embs_t):
        return pl.pallas_call(
            body,
            grid=(FG,),
            in_specs=[pl.BlockSpec((LATENT, FCOLS), lambda j: (0, j))
                      for _ in range(n_tables)],
            out_specs=[pl.BlockSpec((FCOLS, 128), lambda j: (j, 0))
                       for _ in range(n_groups)],
            out_shape=[jax.ShapeDtypeStruct((VP, 128), jnp.float32)
                       for _ in range(n_groups)],
        )(*embs_t)

    return call


_fmt_half0 = _make_fmt(24, (128, 128, 128))
_fmt_half1 = _make_fmt(2, (32,))


def _make_sc(nt):
    ga = (nt + 1) // 2
    gb = nt - ga

    nv = (nt + 7) // 8

    def body(idx_hbm, lidx_hbm, *refs):
        views = refs[0:nv]
        lins = refs[nv:nv + nt]
        s_hbm, q_hbm, l_hbm = refs[nv + nt:nv + nt + 3]
        (idx_v, lidx_v, buf_a, buf_b, lbuf, s_v, q_v, l_v,
         sem_a, sem_b, sem_c) = refs[nv + nt + 3:]

        cid = lax.axis_index("c")
        sid = lax.axis_index("s")
        wid = sid * NUM_CORES + cid
        base = wid * BPW

        pltpu.sync_copy(idx_hbm.at[wid], idx_v)
        pltpu.sync_copy(lidx_hbm.at[wid], lidx_v)

        zero = jnp.zeros((LATENT,), jnp.float32)

        @pl.loop(0, BPW)
        def _(r):
            s_v[r] = zero
            q_v[r] = zero

        @pl.loop(0, BPW // LATENT)
        def _(jj):
            l_v[pl.ds(jj * LATENT, LATENT)] = zero

        def accum_emb(buf, row_base, k):
            @pl.loop(0, k * CHUNK)
            def _(rr):
                v = buf[rr]
                r = row_base + (rr & (CHUNK - 1))
                plsc.addupdate(s_v.at[r], v)
                plsc.addupdate(q_v.at[r], v * v)

        def accum_lin(row_base):
            @pl.loop(0, nt)
            def _(t):
                @pl.loop(0, CHUNK // LATENT)
                def _(jj):
                    seg = pl.ds(jj * LATENT, LATENT)
                    dst = pl.ds(row_base + jj * LATENT, LATENT)
                    plsc.addupdate(l_v.at[dst], lbuf[t, seg])

        @pl.loop(0, NCHUNK)
        def _(j):
            row_base = j * CHUNK
            cps_a = [
                pltpu.async_copy(views[t // 8].at[idx_v.at[t, j]],
                                 buf_a.at[pl.ds(t * CHUNK, CHUNK)], sem_a)
                for t in range(ga)
            ]
            cps_b = [
                pltpu.async_copy(views[(ga + t) // 8].at[idx_v.at[ga + t, j]],
                                 buf_b.at[pl.ds(t * CHUNK, CHUNK)], sem_b)
                for t in range(gb)
            ]
            cps_c = [
                pltpu.async_copy(lins[t].at[lidx_v.at[t, j]], lbuf.at[t],
                                 sem_c)
                for t in range(nt)
            ]
            for c in cps_a:
                c.wait()
            accum_emb(buf_a, row_base, ga)
            for c in cps_b:
                c.wait()
            accum_emb(buf_b, row_base, gb)
            for c in cps_c:
                c.wait()
            accum_lin(row_base)

        pltpu.sync_copy(s_v, s_hbm.at[pl.ds(base, BPW)])
        pltpu.sync_copy(q_v, q_hbm.at[pl.ds(base, BPW)])
        pltpu.sync_copy(l_v, l_hbm.at[pl.ds(base, BPW)])

    return functools.partial(
        pl.kernel,
        out_type=[
            jax.ShapeDtypeStruct((BATCH, LATENT), jnp.float32),
            jax.ShapeDtypeStruct((BATCH, LATENT), jnp.float32),
            jax.ShapeDtypeStruct((BATCH,), jnp.float32),
        ],
        mesh=plsc.VectorSubcoreMesh(core_axis_name="c", subcore_axis_name="s"),
        scratch_types=[
            pltpu.VMEM((nt, NCHUNK, CHUNK), jnp.int32),
            pltpu.VMEM((nt, NCHUNK, CHUNK), jnp.int32),
            pltpu.VMEM((ga * CHUNK, LATENT), jnp.float32),
            pltpu.VMEM((gb * CHUNK, LATENT), jnp.float32),
            pltpu.VMEM((nt, CHUNK), jnp.float32),
            pltpu.VMEM((BPW, LATENT), jnp.float32),
            pltpu.VMEM((BPW, LATENT), jnp.float32),
            pltpu.VMEM((BPW,), jnp.float32),
            pltpu.SemaphoreType.DMA,
            pltpu.SemaphoreType.DMA,
            pltpu.SemaphoreType.DMA,
        ],
        compiler_params=pltpu.CompilerParams(use_tc_tiling_on_sc=False),
    )(body)


_sc_half0 = _make_sc(24)
_sc_half1 = _make_sc(2)


BM = 2048


def _tc_body(dense_ref, s0_ref, s1_ref, q0_ref, q1_ref, l0_ref, l1_ref,
             daw_ref, dab_ref, lw_ref, lb_ref, bias_ref, out_ref):
    d = dense_ref[...]
    demb = jnp.dot(d, daw_ref[...],
                   preferred_element_type=jnp.float32) + dab_ref[...]
    s = s0_ref[...] + s1_ref[...] + demb
    q = q0_ref[...] + q1_ref[...] + demb * demb
    second = 0.5 * (jnp.sum(s * s, axis=1) - jnp.sum(q, axis=1))
    first = (jnp.dot(d, lw_ref[...], preferred_element_type=jnp.float32)[:, 0]
             + lb_ref[0, 0] + l0_ref[...][:, 0] + l1_ref[...][:, 0])
    out_ref[...] = (first + second + bias_ref[0, 0])[:, None]


def _tc_combine(dense, s0, s1, q0, q1, l0, l1, daw, dab, lw, lb, bias):
    grid = BATCH // BM
    bm_spec = pl.BlockSpec((BM, LATENT), lambda i: (i, 0))
    b1_spec = pl.BlockSpec((BM, 1), lambda i: (i, 0))
    return pl.pallas_call(
        _tc_body,
        grid=(grid,),
        in_specs=[
            pl.BlockSpec((BM, NUM_DENSE), lambda i: (i, 0)),
            bm_spec, bm_spec, bm_spec, bm_spec, b1_spec, b1_spec,
            pl.BlockSpec((NUM_DENSE, LATENT), lambda i: (0, 0)),
            pl.BlockSpec((1, LATENT), lambda i: (0, 0)),
            pl.BlockSpec((NUM_DENSE, 1), lambda i: (0, 0)),
            pl.BlockSpec((1, 1), lambda i: (0, 0)),
            pl.BlockSpec((1, 1), lambda i: (0, 0)),
        ],
        out_specs=pl.BlockSpec((BM, 1), lambda i: (i, 0)),
        out_shape=jax.ShapeDtypeStruct((BATCH, 1), jnp.float32),
    )(dense, s0, s1, q0, q1, l0, l1, daw, dab, lw, lb, bias)


def _idx_block(sparses_half, scale8):
    nt = len(sparses_half)
    if scale8:
        arr = jnp.stack([s.astype(jnp.int32) * 8 + (t % 8)
                         for t, s in enumerate(sparses_half)], axis=0)
    else:
        arr = jnp.stack([s.astype(jnp.int32) for s in sparses_half], axis=0)
    return arr.reshape(nt, NW, NCHUNK, CHUNK).transpose(1, 0, 2, 3)


def kernel(dense_0, dense_1, dense_2, dense_3, dense_4, dense_5, dense_6, dense_7, dense_8, dense_9, dense_10, dense_11, dense_12, sparse_0, sparse_1, sparse_2, sparse_3, sparse_4, sparse_5, sparse_6, sparse_7, sparse_8, sparse_9, sparse_10, sparse_11, sparse_12, sparse_13, sparse_14, sparse_15, sparse_16, sparse_17, sparse_18, sparse_19, sparse_20, sparse_21, sparse_22, sparse_23, sparse_24, sparse_25, lin_table_0, lin_table_1, lin_table_2, lin_table_3, lin_table_4, lin_table_5, lin_table_6, lin_table_7, lin_table_8, lin_table_9, lin_table_10, lin_table_11, lin_table_12, lin_table_13, lin_table_14, lin_table_15, lin_table_16, lin_table_17, lin_table_18, lin_table_19, lin_table_20, lin_table_21, lin_table_22, lin_table_23, lin_table_24, lin_table_25, emb_table_0, emb_table_1, emb_table_2, emb_table_3, emb_table_4, emb_table_5, emb_table_6, emb_table_7, emb_table_8, emb_table_9, emb_table_10, emb_table_11, emb_table_12, emb_table_13, emb_table_14, emb_table_15, emb_table_16, emb_table_17, emb_table_18, emb_table_19, emb_table_20, emb_table_21, emb_table_22, emb_table_23, emb_table_24, emb_table_25, lin_dense_w, lin_dense_b, dense_arch_w, dense_arch_b, bias):
    denses = [dense_0, dense_1, dense_2, dense_3, dense_4, dense_5, dense_6,
              dense_7, dense_8, dense_9, dense_10, dense_11, dense_12]
    sparses = [sparse_0, sparse_1, sparse_2, sparse_3, sparse_4, sparse_5,
               sparse_6, sparse_7, sparse_8, sparse_9, sparse_10, sparse_11,
               sparse_12, sparse_13, sparse_14, sparse_15, sparse_16,
               sparse_17, sparse_18, sparse_19, sparse_20, sparse_21,
               sparse_22, sparse_23, sparse_24, sparse_25]
    lin_tables = [lin_table_0, lin_table_1, lin_table_2, lin_table_3,
                  lin_table_4, lin_table_5, lin_table_6, lin_table_7,
                  lin_table_8, lin_table_9, lin_table_10, lin_table_11,
                  lin_table_12, lin_table_13, lin_table_14, lin_table_15,
                  lin_table_16, lin_table_17, lin_table_18, lin_table_19,
                  lin_table_20, lin_table_21, lin_table_22, lin_table_23,
                  lin_table_24, lin_table_25]
    emb_tables = [emb_table_0, emb_table_1, emb_table_2, emb_table_3,
                  emb_table_4, emb_table_5, emb_table_6, emb_table_7,
                  emb_table_8, emb_table_9, emb_table_10, emb_table_11,
                  emb_table_12, emb_table_13, emb_table_14, emb_table_15,
                  emb_table_16, emb_table_17, emb_table_18, emb_table_19,
                  emb_table_20, emb_table_21, emb_table_22, emb_table_23,
                  emb_table_24, emb_table_25]

    embs_t = [jnp.transpose(e) for e in emb_tables]
    lins_lin = [t.reshape(VOCAB_ROWS) for t in lin_tables]

    g0 = _fmt_half0([embs_t[t] for t in HALF0])
    g1 = _fmt_half1([embs_t[t] for t in HALF1])
    v0 = [g.reshape(8 * VP, LATENT) for g in g0]
    v1 = [g.reshape(8 * VP, LATENT) for g in g1]

    e0 = _idx_block([sparses[t] for t in HALF0], True)
    l0i = _idx_block([sparses[t] for t in HALF0], False)
    e1 = _idx_block([sparses[t] for t in HALF1], True)
    l1i = _idx_block([sparses[t] for t in HALF1], False)

    s0, q0, lv0 = _sc_half0(e0, l0i, *v0, *[lins_lin[t] for t in HALF0])
    s1, q1, lv1 = _sc_half1(e1, l1i, *v1, *[lins_lin[t] for t in HALF1])

    dense = jnp.stack(denses, axis=1)
    out = _tc_combine(dense, s0, s1, q0, q1,
                      lv0.reshape(BATCH, 1), lv1.reshape(BATCH, 1),
                      dense_arch_w, dense_arch_b.reshape(1, LATENT),
                      lin_dense_w, lin_dense_b.reshape(1, 1), bias)
    return out


# speedup vs baseline: 1.0451x; 1.0451x over previous
import functools

import jax
import jax.numpy as jnp
from jax import lax
from jax.experimental import pallas as pl
from jax.experimental.pallas import tpu as pltpu
from jax.experimental.pallas import tpu_sc as plsc

NUM_SPARSE = 26
NUM_DENSE = 13
VOCAB_ROWS = 100000
LATENT = 16
BATCH = 16384

NUM_CORES = 2
NUM_SUBCORES = 16
NW = NUM_CORES * NUM_SUBCORES
BPW = BATCH // NW
CHUNK = 128
NCHUNK = BPW // CHUNK

FCOLS = 4096
FG = 25
VP = FG * FCOLS

HALF0 = list(range(22))
HALF1 = list(range(22, 26))


def _make_fmt(n_tables, widths):
    n_groups = len(widths)

    def body(*refs):
        ins = refs[:n_tables]
        outs = refs[n_tables:]
        done = 0
        for g, w in enumerate(widths):
            k = w // 16
            x = jnp.concatenate([ins[done + i][...] for i in range(k)], axis=0)
            if w == 128:
                outs[g][...] = x.T
            else:
                outs[g][:, 0:w] = x.T
            done += k

    def call(embs_t):
        return pl.pallas_call(
            body,
            grid=(FG,),
            in_specs=[pl.BlockSpec((LATENT, FCOLS), lambda j: (0, j))
                      for _ in range(n_tables)],
            out_specs=[pl.BlockSpec((FCOLS, 128), lambda j: (j, 0))
                       for _ in range(n_groups)],
            out_shape=[jax.ShapeDtypeStruct((VP, 128), jnp.float32)
                       for _ in range(n_groups)],
        )(*embs_t)

    return call


_fmt_half0 = _make_fmt(22, (128, 128, 96))
_fmt_half1 = _make_fmt(4, (64,))


def _make_sc(nt):
    ga = (nt + 1) // 2
    gb = nt - ga

    nv = (nt + 7) // 8

    def body(idx_hbm, lidx_hbm, *refs):
        views = refs[0:nv]
        lins = refs[nv:nv + nt]
        s_hbm, q_hbm, l_hbm = refs[nv + nt:nv + nt + 3]
        (idx_v, lidx_v, buf_a, buf_b, lbuf, s_v, q_v, l_v,
         sem_a, sem_b, sem_c) = refs[nv + nt + 3:]

        cid = lax.axis_index("c")
        sid = lax.axis_index("s")
        wid = sid * NUM_CORES + cid
        base = wid * BPW

        pltpu.sync_copy(idx_hbm.at[wid], idx_v)
        pltpu.sync_copy(lidx_hbm.at[wid], lidx_v)

        zero = jnp.zeros((LATENT,), jnp.float32)

        @pl.loop(0, BPW)
        def _(r):
            s_v[r] = zero
            q_v[r] = zero

        @pl.loop(0, BPW // LATENT)
        def _(jj):
            l_v[pl.ds(jj * LATENT, LATENT)] = zero

        def accum_emb(buf, row_base, k):
            @pl.loop(0, k * CHUNK)
            def _(rr):
                v = buf[rr]
                r = row_base + (rr & (CHUNK - 1))
                plsc.addupdate(s_v.at[r], v)
                plsc.addupdate(q_v.at[r], v * v)

        def accum_lin(row_base):
            @pl.loop(0, nt)
            def _(t):
                @pl.loop(0, CHUNK // LATENT)
                def _(jj):
                    seg = pl.ds(jj * LATENT, LATENT)
                    dst = pl.ds(row_base + jj * LATENT, LATENT)
                    plsc.addupdate(l_v.at[dst], lbuf[t, seg])

        @pl.loop(0, NCHUNK)
        def _(j):
            row_base = j * CHUNK
            cps_a = [
                pltpu.async_copy(views[t // 8].at[idx_v.at[t, j]],
                                 buf_a.at[pl.ds(t * CHUNK, CHUNK)], sem_a)
                for t in range(ga)
            ]
            cps_b = [
                pltpu.async_copy(views[(ga + t) // 8].at[idx_v.at[ga + t, j]],
                                 buf_b.at[pl.ds(t * CHUNK, CHUNK)], sem_b)
                for t in range(gb)
            ]
            cps_c = [
                pltpu.async_copy(lins[t].at[lidx_v.at[t, j]], lbuf.at[t],
                                 sem_c)
                for t in range(nt)
            ]
            for c in cps_a:
                c.wait()
            accum_emb(buf_a, row_base, ga)
            for c in cps_b:
                c.wait()
            accum_emb(buf_b, row_base, gb)
            for c in cps_c:
                c.wait()
            accum_lin(row_base)

        pltpu.sync_copy(s_v, s_hbm.at[pl.ds(base, BPW)])
        pltpu.sync_copy(q_v, q_hbm.at[pl.ds(base, BPW)])
        pltpu.sync_copy(l_v, l_hbm.at[pl.ds(base, BPW)])

    return functools.partial(
        pl.kernel,
        out_type=[
            jax.ShapeDtypeStruct((BATCH, LATENT), jnp.float32),
            jax.ShapeDtypeStruct((BATCH, LATENT), jnp.float32),
            jax.ShapeDtypeStruct((BATCH,), jnp.float32),
        ],
        mesh=plsc.VectorSubcoreMesh(core_axis_name="c", subcore_axis_name="s"),
        scratch_types=[
            pltpu.VMEM((nt, NCHUNK, CHUNK), jnp.int32),
            pltpu.VMEM((nt, NCHUNK, CHUNK), jnp.int32),
            pltpu.VMEM((ga * CHUNK, LATENT), jnp.float32),
            pltpu.VMEM((gb * CHUNK, LATENT), jnp.float32),
            pltpu.VMEM((nt, CHUNK), jnp.float32),
            pltpu.VMEM((BPW, LATENT), jnp.float32),
            pltpu.VMEM((BPW, LATENT), jnp.float32),
            pltpu.VMEM((BPW,), jnp.float32),
            pltpu.SemaphoreType.DMA,
            pltpu.SemaphoreType.DMA,
            pltpu.SemaphoreType.DMA,
        ],
        compiler_params=pltpu.CompilerParams(use_tc_tiling_on_sc=False),
    )(body)


_sc_half0 = _make_sc(22)
_sc_half1 = _make_sc(4)


BM = 2048


def _tc_body(dense_ref, s0_ref, s1_ref, q0_ref, q1_ref, l0_ref, l1_ref,
             daw_ref, dab_ref, lw_ref, lb_ref, bias_ref, out_ref):
    d = dense_ref[...]
    demb = jnp.dot(d, daw_ref[...],
                   preferred_element_type=jnp.float32) + dab_ref[...]
    s = s0_ref[...] + s1_ref[...] + demb
    q = q0_ref[...] + q1_ref[...] + demb * demb
    second = 0.5 * (jnp.sum(s * s, axis=1) - jnp.sum(q, axis=1))
    first = (jnp.dot(d, lw_ref[...], preferred_element_type=jnp.float32)[:, 0]
             + lb_ref[0, 0] + l0_ref[...][:, 0] + l1_ref[...][:, 0])
    out_ref[...] = (first + second + bias_ref[0, 0])[:, None]


def _tc_combine(dense, s0, s1, q0, q1, l0, l1, daw, dab, lw, lb, bias):
    grid = BATCH // BM
    bm_spec = pl.BlockSpec((BM, LATENT), lambda i: (i, 0))
    b1_spec = pl.BlockSpec((BM, 1), lambda i: (i, 0))
    return pl.pallas_call(
        _tc_body,
        grid=(grid,),
        in_specs=[
            pl.BlockSpec((BM, NUM_DENSE), lambda i: (i, 0)),
            bm_spec, bm_spec, bm_spec, bm_spec, b1_spec, b1_spec,
            pl.BlockSpec((NUM_DENSE, LATENT), lambda i: (0, 0)),
            pl.BlockSpec((1, LATENT), lambda i: (0, 0)),
            pl.BlockSpec((NUM_DENSE, 1), lambda i: (0, 0)),
            pl.BlockSpec((1, 1), lambda i: (0, 0)),
            pl.BlockSpec((1, 1), lambda i: (0, 0)),
        ],
        out_specs=pl.BlockSpec((BM, 1), lambda i: (i, 0)),
        out_shape=jax.ShapeDtypeStruct((BATCH, 1), jnp.float32),
    )(dense, s0, s1, q0, q1, l0, l1, daw, dab, lw, lb, bias)


def _idx_block(sparses_half, scale8):
    nt = len(sparses_half)
    if scale8:
        arr = jnp.stack([s.astype(jnp.int32) * 8 + (t % 8)
                         for t, s in enumerate(sparses_half)], axis=0)
    else:
        arr = jnp.stack([s.astype(jnp.int32) for s in sparses_half], axis=0)
    return arr.reshape(nt, NW, NCHUNK, CHUNK).transpose(1, 0, 2, 3)


def kernel(dense_0, dense_1, dense_2, dense_3, dense_4, dense_5, dense_6, dense_7, dense_8, dense_9, dense_10, dense_11, dense_12, sparse_0, sparse_1, sparse_2, sparse_3, sparse_4, sparse_5, sparse_6, sparse_7, sparse_8, sparse_9, sparse_10, sparse_11, sparse_12, sparse_13, sparse_14, sparse_15, sparse_16, sparse_17, sparse_18, sparse_19, sparse_20, sparse_21, sparse_22, sparse_23, sparse_24, sparse_25, lin_table_0, lin_table_1, lin_table_2, lin_table_3, lin_table_4, lin_table_5, lin_table_6, lin_table_7, lin_table_8, lin_table_9, lin_table_10, lin_table_11, lin_table_12, lin_table_13, lin_table_14, lin_table_15, lin_table_16, lin_table_17, lin_table_18, lin_table_19, lin_table_20, lin_table_21, lin_table_22, lin_table_23, lin_table_24, lin_table_25, emb_table_0, emb_table_1, emb_table_2, emb_table_3, emb_table_4, emb_table_5, emb_table_6, emb_table_7, emb_table_8, emb_table_9, emb_table_10, emb_table_11, emb_table_12, emb_table_13, emb_table_14, emb_table_15, emb_table_16, emb_table_17, emb_table_18, emb_table_19, emb_table_20, emb_table_21, emb_table_22, emb_table_23, emb_table_24, emb_table_25, lin_dense_w, lin_dense_b, dense_arch_w, dense_arch_b, bias):
    denses = [dense_0, dense_1, dense_2, dense_3, dense_4, dense_5, dense_6,
              dense_7, dense_8, dense_9, dense_10, dense_11, dense_12]
    sparses = [sparse_0, sparse_1, sparse_2, sparse_3, sparse_4, sparse_5,
               sparse_6, sparse_7, sparse_8, sparse_9, sparse_10, sparse_11,
               sparse_12, sparse_13, sparse_14, sparse_15, sparse_16,
               sparse_17, sparse_18, sparse_19, sparse_20, sparse_21,
               sparse_22, sparse_23, sparse_24, sparse_25]
    lin_tables = [lin_table_0, lin_table_1, lin_table_2, lin_table_3,
                  lin_table_4, lin_table_5, lin_table_6, lin_table_7,
                  lin_table_8, lin_table_9, lin_table_10, lin_table_11,
                  lin_table_12, lin_table_13, lin_table_14, lin_table_15,
                  lin_table_16, lin_table_17, lin_table_18, lin_table_19,
                  lin_table_20, lin_table_21, lin_table_22, lin_table_23,
                  lin_table_24, lin_table_25]
    emb_tables = [emb_table_0, emb_table_1, emb_table_2, emb_table_3,
                  emb_table_4, emb_table_5, emb_table_6, emb_table_7,
                  emb_table_8, emb_table_9, emb_table_10, emb_table_11,
                  emb_table_12, emb_table_13, emb_table_14, emb_table_15,
                  emb_table_16, emb_table_17, emb_table_18, emb_table_19,
                  emb_table_20, emb_table_21, emb_table_22, emb_table_23,
                  emb_table_24, emb_table_25]

    embs_t = [jnp.transpose(e) for e in emb_tables]
    lins_lin = [t.reshape(VOCAB_ROWS) for t in lin_tables]

    g0 = _fmt_half0([embs_t[t] for t in HALF0])
    g1 = _fmt_half1([embs_t[t] for t in HALF1])
    v0 = [g.reshape(8 * VP, LATENT) for g in g0]
    v1 = [g.reshape(8 * VP, LATENT) for g in g1]

    e0 = _idx_block([sparses[t] for t in HALF0], True)
    l0i = _idx_block([sparses[t] for t in HALF0], False)
    e1 = _idx_block([sparses[t] for t in HALF1], True)
    l1i = _idx_block([sparses[t] for t in HALF1], False)

    s0, q0, lv0 = _sc_half0(e0, l0i, *v0, *[lins_lin[t] for t in HALF0])
    s1, q1, lv1 = _sc_half1(e1, l1i, *v1, *[lins_lin[t] for t in HALF1])

    dense = jnp.stack(denses, axis=1)
    out = _tc_combine(dense, s0, s1, q0, q1,
                      lv0.reshape(BATCH, 1), lv1.reshape(BATCH, 1),
                      dense_arch_w, dense_arch_b.reshape(1, LATENT),
                      lin_dense_w, lin_dense_b.reshape(1, 1), bias)
    return out


# speedup vs baseline: 1.0890x; 1.0419x over previous
import functools

import jax
import jax.numpy as jnp
from jax import lax
from jax.experimental import pallas as pl
from jax.experimental.pallas import tpu as pltpu
from jax.experimental.pallas import tpu_sc as plsc

NUM_SPARSE = 26
NUM_DENSE = 13
VOCAB_ROWS = 100000
LATENT = 16
BATCH = 16384

NUM_CORES = 2
NUM_SUBCORES = 16
NW = NUM_CORES * NUM_SUBCORES
BPW = BATCH // NW
CHUNK = 128
NCHUNK = BPW // CHUNK

FCOLS = 4096
FG = 25
VP = FG * FCOLS

HALF0 = list(range(20))
HALF1 = list(range(20, 26))


def _make_fmt(n_tables, widths):
    n_groups = len(widths)

    def body(*refs):
        ins = refs[:n_tables]
        outs = refs[n_tables:]
        done = 0
        for g, w in enumerate(widths):
            k = w // 16
            x = jnp.concatenate([ins[done + i][...] for i in range(k)], axis=0)
            if w == 128:
                outs[g][...] = x.T
            else:
                outs[g][:, 0:w] = x.T
            done += k

    def call(embs_t):
        return pl.pallas_call(
            body,
            grid=(FG,),
            in_specs=[pl.BlockSpec((LATENT, FCOLS), lambda j: (0, j))
                      for _ in range(n_tables)],
            out_specs=[pl.BlockSpec((FCOLS, 128), lambda j: (j, 0))
                       for _ in range(n_groups)],
            out_shape=[jax.ShapeDtypeStruct((VP, 128), jnp.float32)
                       for _ in range(n_groups)],
        )(*embs_t)

    return call


_fmt_half0 = _make_fmt(20, (128, 128, 64))
_fmt_half1 = _make_fmt(6, (96,))


def _make_sc(nt):
    ga = (nt + 1) // 2
    gb = nt - ga

    nv = (nt + 7) // 8

    def body(idx_hbm, lidx_hbm, *refs):
        views = refs[0:nv]
        lins = refs[nv:nv + nt]
        s_hbm, q_hbm, l_hbm = refs[nv + nt:nv + nt + 3]
        (idx_v, lidx_v, buf_a, buf_b, lbuf, s_v, q_v, l_v,
         sem_a, sem_b, sem_c) = refs[nv + nt + 3:]

        cid = lax.axis_index("c")
        sid = lax.axis_index("s")
        wid = sid * NUM_CORES + cid
        base = wid * BPW

        pltpu.sync_copy(idx_hbm.at[wid], idx_v)
        pltpu.sync_copy(lidx_hbm.at[wid], lidx_v)

        zero = jnp.zeros((LATENT,), jnp.float32)

        @pl.loop(0, BPW)
        def _(r):
            s_v[r] = zero
            q_v[r] = zero

        @pl.loop(0, BPW // LATENT)
        def _(jj):
            l_v[pl.ds(jj * LATENT, LATENT)] = zero

        def accum_emb(buf, row_base, k):
            @pl.loop(0, k * CHUNK)
            def _(rr):
                v = buf[rr]
                r = row_base + (rr & (CHUNK - 1))
                plsc.addupdate(s_v.at[r], v)
                plsc.addupdate(q_v.at[r], v * v)

        def accum_lin(row_base):
            @pl.loop(0, nt)
            def _(t):
                @pl.loop(0, CHUNK // LATENT)
                def _(jj):
                    seg = pl.ds(jj * LATENT, LATENT)
                    dst = pl.ds(row_base + jj * LATENT, LATENT)
                    plsc.addupdate(l_v.at[dst], lbuf[t, seg])

        @pl.loop(0, NCHUNK)
        def _(j):
            row_base = j * CHUNK
            cps_a = [
                pltpu.async_copy(views[t // 8].at[idx_v.at[t, j]],
                                 buf_a.at[pl.ds(t * CHUNK, CHUNK)], sem_a)
                for t in range(ga)
            ]
            cps_b = [
                pltpu.async_copy(views[(ga + t) // 8].at[idx_v.at[ga + t, j]],
                                 buf_b.at[pl.ds(t * CHUNK, CHUNK)], sem_b)
                for t in range(gb)
            ]
            cps_c = [
                pltpu.async_copy(lins[t].at[lidx_v.at[t, j]], lbuf.at[t],
                                 sem_c)
                for t in range(nt)
            ]
            for c in cps_a:
                c.wait()
            accum_emb(buf_a, row_base, ga)
            for c in cps_b:
                c.wait()
            accum_emb(buf_b, row_base, gb)
            for c in cps_c:
                c.wait()
            accum_lin(row_base)

        pltpu.sync_copy(s_v, s_hbm.at[pl.ds(base, BPW)])
        pltpu.sync_copy(q_v, q_hbm.at[pl.ds(base, BPW)])
        pltpu.sync_copy(l_v, l_hbm.at[pl.ds(base, BPW)])

    return functools.partial(
        pl.kernel,
        out_type=[
            jax.ShapeDtypeStruct((BATCH, LATENT), jnp.float32),
            jax.ShapeDtypeStruct((BATCH, LATENT), jnp.float32),
            jax.ShapeDtypeStruct((BATCH,), jnp.float32),
        ],
        mesh=plsc.VectorSubcoreMesh(core_axis_name="c", subcore_axis_name="s"),
        scratch_types=[
            pltpu.VMEM((nt, NCHUNK, CHUNK), jnp.int32),
            pltpu.VMEM((nt, NCHUNK, CHUNK), jnp.int32),
            pltpu.VMEM((ga * CHUNK, LATENT), jnp.float32),
            pltpu.VMEM((gb * CHUNK, LATENT), jnp.float32),
            pltpu.VMEM((nt, CHUNK), jnp.float32),
            pltpu.VMEM((BPW, LATENT), jnp.float32),
            pltpu.VMEM((BPW, LATENT), jnp.float32),
            pltpu.VMEM((BPW,), jnp.float32),
            pltpu.SemaphoreType.DMA,
            pltpu.SemaphoreType.DMA,
            pltpu.SemaphoreType.DMA,
        ],
        compiler_params=pltpu.CompilerParams(use_tc_tiling_on_sc=False),
    )(body)


_sc_half0 = _make_sc(20)
_sc_half1 = _make_sc(6)


BM = 2048


def _tc_body(dense_ref, s0_ref, s1_ref, q0_ref, q1_ref, l0_ref, l1_ref,
             daw_ref, dab_ref, lw_ref, lb_ref, bias_ref, out_ref):
    d = dense_ref[...]
    demb = jnp.dot(d, daw_ref[...],
                   preferred_element_type=jnp.float32) + dab_ref[...]
    s = s0_ref[...] + s1_ref[...] + demb
    q = q0_ref[...] + q1_ref[...] + demb * demb
    second = 0.5 * (jnp.sum(s * s, axis=1) - jnp.sum(q, axis=1))
    first = (jnp.dot(d, lw_ref[...], preferred_element_type=jnp.float32)[:, 0]
             + lb_ref[0, 0] + l0_ref[...][:, 0] + l1_ref[...][:, 0])
    out_ref[...] = (first + second + bias_ref[0, 0])[:, None]


def _tc_combine(dense, s0, s1, q0, q1, l0, l1, daw, dab, lw, lb, bias):
    grid = BATCH // BM
    bm_spec = pl.BlockSpec((BM, LATENT), lambda i: (i, 0))
    b1_spec = pl.BlockSpec((BM, 1), lambda i: (i, 0))
    return pl.pallas_call(
        _tc_body,
        grid=(grid,),
        in_specs=[
            pl.BlockSpec((BM, NUM_DENSE), lambda i: (i, 0)),
            bm_spec, bm_spec, bm_spec, bm_spec, b1_spec, b1_spec,
            pl.BlockSpec((NUM_DENSE, LATENT), lambda i: (0, 0)),
            pl.BlockSpec((1, LATENT), lambda i: (0, 0)),
            pl.BlockSpec((NUM_DENSE, 1), lambda i: (0, 0)),
            pl.BlockSpec((1, 1), lambda i: (0, 0)),
            pl.BlockSpec((1, 1), lambda i: (0, 0)),
        ],
        out_specs=pl.BlockSpec((BM, 1), lambda i: (i, 0)),
        out_shape=jax.ShapeDtypeStruct((BATCH, 1), jnp.float32),
    )(dense, s0, s1, q0, q1, l0, l1, daw, dab, lw, lb, bias)


def _idx_block(sparses_half, scale8):
    nt = len(sparses_half)
    if scale8:
        arr = jnp.stack([s.astype(jnp.int32) * 8 + (t % 8)
                         for t, s in enumerate(sparses_half)], axis=0)
    else:
        arr = jnp.stack([s.astype(jnp.int32) for s in sparses_half], axis=0)
    return arr.reshape(nt, NW, NCHUNK, CHUNK).transpose(1, 0, 2, 3)


def kernel(dense_0, dense_1, dense_2, dense_3, dense_4, dense_5, dense_6, dense_7, dense_8, dense_9, dense_10, dense_11, dense_12, sparse_0, sparse_1, sparse_2, sparse_3, sparse_4, sparse_5, sparse_6, sparse_7, sparse_8, sparse_9, sparse_10, sparse_11, sparse_12, sparse_13, sparse_14, sparse_15, sparse_16, sparse_17, sparse_18, sparse_19, sparse_20, sparse_21, sparse_22, sparse_23, sparse_24, sparse_25, lin_table_0, lin_table_1, lin_table_2, lin_table_3, lin_table_4, lin_table_5, lin_table_6, lin_table_7, lin_table_8, lin_table_9, lin_table_10, lin_table_11, lin_table_12, lin_table_13, lin_table_14, lin_table_15, lin_table_16, lin_table_17, lin_table_18, lin_table_19, lin_table_20, lin_table_21, lin_table_22, lin_table_23, lin_table_24, lin_table_25, emb_table_0, emb_table_1, emb_table_2, emb_table_3, emb_table_4, emb_table_5, emb_table_6, emb_table_7, emb_table_8, emb_table_9, emb_table_10, emb_table_11, emb_table_12, emb_table_13, emb_table_14, emb_table_15, emb_table_16, emb_table_17, emb_table_18, emb_table_19, emb_table_20, emb_table_21, emb_table_22, emb_table_23, emb_table_24, emb_table_25, lin_dense_w, lin_dense_b, dense_arch_w, dense_arch_b, bias):
    denses = [dense_0, dense_1, dense_2, dense_3, dense_4, dense_5, dense_6,
              dense_7, dense_8, dense_9, dense_10, dense_11, dense_12]
    sparses = [sparse_0, sparse_1, sparse_2, sparse_3, sparse_4, sparse_5,
               sparse_6, sparse_7, sparse_8, sparse_9, sparse_10, sparse_11,
               sparse_12, sparse_13, sparse_14, sparse_15, sparse_16,
               sparse_17, sparse_18, sparse_19, sparse_20, sparse_21,
               sparse_22, sparse_23, sparse_24, sparse_25]
    lin_tables = [lin_table_0, lin_table_1, lin_table_2, lin_table_3,
                  lin_table_4, lin_table_5, lin_table_6, lin_table_7,
                  lin_table_8, lin_table_9, lin_table_10, lin_table_11,
                  lin_table_12, lin_table_13, lin_table_14, lin_table_15,
                  lin_table_16, lin_table_17, lin_table_18, lin_table_19,
                  lin_table_20, lin_table_21, lin_table_22, lin_table_23,
                  lin_table_24, lin_table_25]
    emb_tables = [emb_table_0, emb_table_1, emb_table_2, emb_table_3,
                  emb_table_4, emb_table_5, emb_table_6, emb_table_7,
                  emb_table_8, emb_table_9, emb_table_10, emb_table_11,
                  emb_table_12, emb_table_13, emb_table_14, emb_table_15,
                  emb_table_16, emb_table_17, emb_table_18, emb_table_19,
                  emb_table_20, emb_table_21, emb_table_22, emb_table_23,
                  emb_table_24, emb_table_25]

    embs_t = [jnp.transpose(e) for e in emb_tables]
    lins_lin = [t.reshape(VOCAB_ROWS) for t in lin_tables]

    g0 = _fmt_half0([embs_t[t] for t in HALF0])
    g1 = _fmt_half1([embs_t[t] for t in HALF1])
    v0 = [g.reshape(8 * VP, LATENT) for g in g0]
    v1 = [g.reshape(8 * VP, LATENT) for g in g1]

    e0 = _idx_block([sparses[t] for t in HALF0], True)
    l0i = _idx_block([sparses[t] for t in HALF0], False)
    e1 = _idx_block([sparses[t] for t in HALF1], True)
    l1i = _idx_block([sparses[t] for t in HALF1], False)

    s0, q0, lv0 = _sc_half0(e0, l0i, *v0, *[lins_lin[t] for t in HALF0])
    s1, q1, lv1 = _sc_half1(e1, l1i, *v1, *[lins_lin[t] for t in HALF1])

    dense = jnp.stack(denses, axis=1)
    out = _tc_combine(dense, s0, s1, q0, q1,
                      lv0.reshape(BATCH, 1), lv1.reshape(BATCH, 1),
                      dense_arch_w, dense_arch_b.reshape(1, LATENT),
                      lin_dense_w, lin_dense_b.reshape(1, 1), bias)
    return out
